# Initial kernel scaffold; baseline (speedup 1.0000x reference)
import jax
import jax.numpy as jnp
from jax.experimental import pallas as pl


def kernel(edge_index, u_emb, i_emb, W1, b1, W2, b2):
    raise NotImplementedError("write your pallas kernel here")


# speedup vs baseline: 32.6718x; 32.6718x over previous
import functools

import jax
import jax.numpy as jnp
from jax import lax
from jax.experimental import pallas as pl
from jax.experimental.pallas import tpu as pltpu
from jax.experimental.pallas import tpu_sc as plsc


def _sc_degree_counts(dst2d, zeros_npad, n_pad, num_edges):
    info = plsc.get_sparse_core_info()
    nc, ns = info.num_cores, info.num_subcores
    nw = nc * ns
    k = dst2d.shape[1]
    e_per_w = num_edges // nw
    nch = e_per_w // k
    n_per_s = n_pad // ns
    mesh = plsc.VectorSubcoreMesh(core_axis_name="c", subcore_axis_name="s")

    @functools.partial(
        pl.kernel,
        out_type=jax.ShapeDtypeStruct((nc * n_pad,), jnp.float32),
        mesh=mesh,
        scratch_types=[
            pltpu.VMEM((nch, k), jnp.int32),
            pltpu.VMEM((k,), jnp.float32),
            pltpu.VMEM((n_per_s,), jnp.float32),
            pltpu.VMEM_SHARED((n_pad,), jnp.float32),
            pltpu.SemaphoreType.DMA,
        ],
        compiler_params=pltpu.CompilerParams(use_tc_tiling_on_sc=False),
    )
    def body(dst_hbm, zeros_hbm, out_hbm, idx_v, ones_v, bounce, acc, sem):
        c = lax.axis_index("c")
        s = lax.axis_index("s")
        wid = s * nc + c
        for i in range(k // 16):
            ones_v[pl.ds(i * 16, 16)] = jnp.full((16,), 1.0, jnp.float32)
        pltpu.sync_copy(dst_hbm.at[pl.ds(wid * nch, nch)], idx_v)
        pltpu.sync_copy(zeros_hbm.at[pl.ds(s * n_per_s, n_per_s)], bounce)
        pltpu.sync_copy(bounce, acc.at[pl.ds(s * n_per_s, n_per_s)])
        plsc.subcore_barrier()
        grp = 25
        for g0 in range(0, nch, grp):
            descs = [pltpu.async_copy(ones_v, acc.at[idx_v.at[j]], sem,
                                      add=True)
                     for j in range(g0, g0 + grp)]
            for de in descs:
                de.wait()
        plsc.subcore_barrier()
        pltpu.sync_copy(acc.at[pl.ds(s * n_per_s, n_per_s)], bounce)
        pltpu.sync_copy(bounce,
                        out_hbm.at[pl.ds(c * n_pad + s * n_per_s, n_per_s)])

    return body(dst2d, zeros_npad)


def _sc_scatter_rows(g, src2d, dst2d, zeros_nd, n_pad, num_edges):
    d = g.shape[1]
    dt = g.dtype
    info = plsc.get_sparse_core_info()
    nc, ns = info.num_cores, info.num_subcores
    k = src2d.shape[1]
    nch = num_edges // (nc * ns) // k
    nbuf = 5
    la = 3
    n_per_s = n_pad // ns
    mesh = plsc.VectorSubcoreMesh(core_axis_name="c", subcore_axis_name="s")

    @functools.partial(
        pl.kernel,
        out_type=jax.ShapeDtypeStruct((nc * n_pad, d), dt),
        mesh=mesh,
        scratch_types=[
            pltpu.VMEM((nch, k), jnp.int32),
            pltpu.VMEM((nch, k), jnp.int32),
            [pltpu.VMEM((k, d), dt) for _ in range(nbuf)],
            pltpu.VMEM((n_per_s, d), dt),
            pltpu.VMEM_SHARED((n_pad, d), dt),
            [pltpu.SemaphoreType.DMA for _ in range(nbuf)],
            [pltpu.SemaphoreType.DMA for _ in range(nbuf)],
        ],
        compiler_params=pltpu.CompilerParams(use_tc_tiling_on_sc=False),
    )
    def body(g_hbm, src_hbm, dst_hbm, zeros_hbm, out_hbm,
             src_v, dst_v, rows, bounce, acc, gsem, ssem):
        c = lax.axis_index("c")
        s = lax.axis_index("s")
        wid = c * ns + s
        pltpu.sync_copy(src_hbm.at[pl.ds(wid * nch, nch)], src_v)
        pltpu.sync_copy(dst_hbm.at[pl.ds(wid * nch, nch)], dst_v)
        pltpu.sync_copy(zeros_hbm, bounce)
        pltpu.sync_copy(bounce, acc.at[pl.ds(s * n_per_s, n_per_s)])
        plsc.subcore_barrier()

        def gather_start(b, j):
            pltpu.async_copy(g_hbm.at[src_v.at[j]], rows[b], gsem[b])

        def gather_wait(b):
            pltpu.make_async_copy(g_hbm.at[src_v.at[0]], rows[b],
                                  gsem[b]).wait()

        def scatter_start(b, j):
            pltpu.async_copy(rows[b], acc.at[dst_v.at[j]], ssem[b], add=True)

        def scatter_wait(b):
            pltpu.make_async_copy(rows[b], acc.at[dst_v.at[0]], ssem[b]).wait()

        def do_chunk(b, j, prefetch):
            gather_wait(b)
            scatter_start(b, j)
            if prefetch:
                b2 = (b + la) % nbuf
                scatter_wait(b2)
                gather_start(b2, j + la)

        for j in range(la):
            gather_start(j % nbuf, j)
        for j in range(nbuf - la):
            b2 = (j + la) % nbuf
            gather_wait(j % nbuf)
            scatter_start(j % nbuf, j)
            gather_start(b2, j + la)
        start = nbuf - la
        ngroups = (nch - la - start) // nbuf

        def group(gi, carry):
            j0 = start + gi * nbuf
            for t in range(nbuf):
                do_chunk((start + t) % nbuf, j0 + t, True)
            return carry

        lax.fori_loop(0, ngroups, group, 0)
        for j in range(nch - la, nch):
            do_chunk(j % nbuf, j, False)
        for b in range(nbuf):
            scatter_wait(b)

        plsc.subcore_barrier()
        pltpu.sync_copy(acc.at[pl.ds(s * n_per_s, n_per_s)], bounce)
        pltpu.sync_copy(bounce,
                        out_hbm.at[pl.ds(c * n_pad + s * n_per_s, n_per_s)])

    return body(g, src2d, dst2d, zeros_nd)


def _dis_from_counts(c_blk):
    deg = c_blk[0] + c_blk[1] + 1.0
    return lax.rsqrt(deg)


def _tc_scale_matmul(x0, w1, counts2):
    n, d = x0.shape
    r = 1000
    grid = n // r

    def body(x_ref, w_ref, c_ref, g_ref):
        dis = _dis_from_counts(c_ref[...])
        h = lax.dot_general(x_ref[...], w_ref[...],
                            (((1,), (1,)), ((), ())),
                            preferred_element_type=jnp.float32)
        g_ref[...] = (h * dis).astype(jnp.bfloat16)

    return pl.pallas_call(
        body,
        out_shape=jax.ShapeDtypeStruct((n, d), jnp.bfloat16),
        grid=(grid,),
        in_specs=[
            pl.BlockSpec((r, d), lambda i: (i, 0)),
            pl.BlockSpec((d, d), lambda i: (0, 0)),
            pl.BlockSpec((2, r, 1), lambda i: (0, i, 0)),
        ],
        out_specs=pl.BlockSpec((r, d), lambda i: (i, 0)),
    )(x0, w1, counts2)


def _tc_finish_matmul(agg2, g1, counts2, b1, w2):
    n, d = g1.shape
    r = 1000
    grid = n // r

    def body(a_ref, g_ref, c_ref, b_ref, w_ref, h1_ref, g2_ref):
        dis = _dis_from_counts(c_ref[...])
        a = (a_ref[0].astype(jnp.float32) + a_ref[1].astype(jnp.float32)
             + g_ref[...].astype(jnp.float32))
        h1 = a * dis + b_ref[...]
        h1_ref[...] = h1
        h2l = lax.dot_general(h1, w_ref[...],
                              (((1,), (1,)), ((), ())),
                              preferred_element_type=jnp.float32)
        g2_ref[...] = (h2l * dis).astype(jnp.bfloat16)

    return pl.pallas_call(
        body,
        out_shape=(jax.ShapeDtypeStruct((n, d), jnp.float32),
                   jax.ShapeDtypeStruct((n, d), jnp.bfloat16)),
        grid=(grid,),
        in_specs=[
            pl.BlockSpec((2, r, d), lambda i: (0, i, 0)),
            pl.BlockSpec((r, d), lambda i: (i, 0)),
            pl.BlockSpec((2, r, 1), lambda i: (0, i, 0)),
            pl.BlockSpec((1, d), lambda i: (0, 0)),
            pl.BlockSpec((d, d), lambda i: (0, 0)),
        ],
        out_specs=(pl.BlockSpec((r, d), lambda i: (i, 0)),
                   pl.BlockSpec((r, d), lambda i: (i, 0))),
    )(agg2, g1, counts2, b1, w2)


def _tc_final(agg2, g2, counts2, b2, x0, h1):
    n, d = g2.shape
    r = 1000
    grid = n // r

    def body(a_ref, g_ref, c_ref, b_ref, x_ref, h1_ref, o_ref):
        dis = _dis_from_counts(c_ref[...])
        a = (a_ref[0].astype(jnp.float32) + a_ref[1].astype(jnp.float32)
             + g_ref[...].astype(jnp.float32))
        h2 = a * dis + b_ref[...]
        o_ref[...] = (x_ref[...] + h1_ref[...] + h2) * (1.0 / 3.0)

    return pl.pallas_call(
        body,
        out_shape=jax.ShapeDtypeStruct((n, d), jnp.float32),
        grid=(grid,),
        in_specs=[
            pl.BlockSpec((2, r, d), lambda i: (0, i, 0)),
            pl.BlockSpec((r, d), lambda i: (i, 0)),
            pl.BlockSpec((2, r, 1), lambda i: (0, i, 0)),
            pl.BlockSpec((1, d), lambda i: (0, 0)),
            pl.BlockSpec((r, d), lambda i: (i, 0)),
            pl.BlockSpec((r, d), lambda i: (i, 0)),
        ],
        out_specs=pl.BlockSpec((r, d), lambda i: (i, 0)),
    )(agg2, g2, counts2, b2, x0, h1)


def kernel(edge_index, u_emb, i_emb, W1, b1, W2, b2):
    n = u_emb.shape[0] + i_emb.shape[0]
    d = u_emb.shape[1]
    num_edges = edge_index.shape[1]
    ns = 16
    n_pad = ((n + 8 * ns - 1) // (8 * ns)) * (8 * ns)

    k = 80
    src2d = edge_index[0].reshape(num_edges // k, k)
    dst2d = edge_index[1].reshape(num_edges // k, k)
    x0 = jnp.concatenate([u_emb, i_emb], axis=0)
    b1r = b1.reshape(1, d)
    b2r = b2.reshape(1, d)
    zeros_n = jnp.zeros((n_pad,), jnp.float32)
    zeros_nd = jnp.zeros((n_pad // 16, d), jnp.bfloat16)

    counts = _sc_degree_counts(dst2d, zeros_n, n_pad, num_edges)
    counts2 = counts.reshape(2, n_pad, 1)

    g1 = _tc_scale_matmul(x0, W1, counts2)
    agg1 = _sc_scatter_rows(g1, src2d, dst2d, zeros_nd, n_pad, num_edges)
    agg1 = agg1.reshape(2, n_pad, d)

    h1, g2 = _tc_finish_matmul(agg1, g1, counts2, b1r, W2)
    agg2 = _sc_scatter_rows(g2, src2d, dst2d, zeros_nd, n_pad, num_edges)
    agg2 = agg2.reshape(2, n_pad, d)

    return _tc_final(agg2, g2, counts2, b2r, x0, h1)


# speedup vs baseline: 34.0943x; 1.0435x over previous
import functools

import jax
import jax.numpy as jnp
from jax import lax
from jax.experimental import pallas as pl
from jax.experimental.pallas import tpu as pltpu
from jax.experimental.pallas import tpu_sc as plsc


def _sc_degree_counts(dst2d, zeros_npad, n_pad, num_edges):
    info = plsc.get_sparse_core_info()
    nc, ns = info.num_cores, info.num_subcores
    nw = nc * ns
    k = dst2d.shape[1]
    e_per_w = num_edges // nw
    nch = e_per_w // k
    n_per_s = n_pad // ns
    mesh = plsc.VectorSubcoreMesh(core_axis_name="c", subcore_axis_name="s")

    @functools.partial(
        pl.kernel,
        out_type=jax.ShapeDtypeStruct((nc * n_pad,), jnp.float32),
        mesh=mesh,
        scratch_types=[
            pltpu.VMEM((nch, k), jnp.int32),
            pltpu.VMEM((k,), jnp.float32),
            pltpu.VMEM((n_per_s,), jnp.float32),
            pltpu.VMEM_SHARED((n_pad,), jnp.float32),
            pltpu.SemaphoreType.DMA,
        ],
        compiler_params=pltpu.CompilerParams(use_tc_tiling_on_sc=False),
    )
    def body(dst_hbm, zeros_hbm, out_hbm, idx_v, ones_v, bounce, acc, sem):
        c = lax.axis_index("c")
        s = lax.axis_index("s")
        wid = s * nc + c
        for i in range(k // 16):
            ones_v[pl.ds(i * 16, 16)] = jnp.full((16,), 1.0, jnp.float32)
        pltpu.sync_copy(dst_hbm.at[pl.ds(wid * nch, nch)], idx_v)
        pltpu.sync_copy(zeros_hbm.at[pl.ds(s * n_per_s, n_per_s)], bounce)
        pltpu.sync_copy(bounce, acc.at[pl.ds(s * n_per_s, n_per_s)])
        plsc.subcore_barrier()
        grp = 25
        for g0 in range(0, nch, grp):
            descs = [pltpu.async_copy(ones_v, acc.at[idx_v.at[j]], sem,
                                      add=True)
                     for j in range(g0, g0 + grp)]
            for de in descs:
                de.wait()
        plsc.subcore_barrier()
        pltpu.sync_copy(acc.at[pl.ds(s * n_per_s, n_per_s)], bounce)
        pltpu.sync_copy(bounce,
                        out_hbm.at[pl.ds(c * n_pad + s * n_per_s, n_per_s)])

    return body(dst2d, zeros_npad)


def _sc_scatter_rows(g, src2d, dst2d, zeros_nd, n_pad, num_edges):
    d = g.shape[1]
    dt = g.dtype
    info = plsc.get_sparse_core_info()
    nc, ns = info.num_cores, info.num_subcores
    k = src2d.shape[1]
    nch = num_edges // (nc * ns) // k
    nbuf = 5
    la = 3
    n_per_s = n_pad // ns
    mesh = plsc.VectorSubcoreMesh(core_axis_name="c", subcore_axis_name="s")

    @functools.partial(
        pl.kernel,
        out_type=jax.ShapeDtypeStruct((nc * n_pad, d), dt),
        mesh=mesh,
        scratch_types=[
            pltpu.VMEM((nch, k), jnp.int32),
            pltpu.VMEM((nch, k), jnp.int32),
            [pltpu.VMEM((k, d), dt) for _ in range(nbuf)],
            pltpu.VMEM((n_per_s // 4, d), dt),
            pltpu.VMEM_SHARED((n_pad, d), dt),
            [pltpu.SemaphoreType.DMA for _ in range(nbuf)],
            [pltpu.SemaphoreType.DMA for _ in range(nbuf)],
        ],
        compiler_params=pltpu.CompilerParams(use_tc_tiling_on_sc=False),
    )
    def body(g_hbm, src_hbm, dst_hbm, zeros_hbm, out_hbm,
             src_v, dst_v, rows, bounce, acc, gsem, ssem):
        c = lax.axis_index("c")
        s = lax.axis_index("s")
        wid = c * ns + s
        pltpu.sync_copy(src_hbm.at[pl.ds(wid * nch, nch)], src_v)
        pltpu.sync_copy(dst_hbm.at[pl.ds(wid * nch, nch)], dst_v)
        q = n_per_s // 4
        pltpu.sync_copy(zeros_hbm.at[pl.ds(0, q)], bounce)
        for t in range(4):
            pltpu.sync_copy(bounce, acc.at[pl.ds(s * n_per_s + t * q, q)])
        plsc.subcore_barrier()

        def gather_start(b, j):
            pltpu.async_copy(g_hbm.at[src_v.at[j]], rows[b], gsem[b])

        def gather_wait(b):
            pltpu.make_async_copy(g_hbm.at[src_v.at[0]], rows[b],
                                  gsem[b]).wait()

        def scatter_start(b, j):
            pltpu.async_copy(rows[b], acc.at[dst_v.at[j]], ssem[b], add=True)

        def scatter_wait(b):
            pltpu.make_async_copy(rows[b], acc.at[dst_v.at[0]], ssem[b]).wait()

        def do_chunk(b, j, prefetch):
            gather_wait(b)
            scatter_start(b, j)
            if prefetch:
                b2 = (b + la) % nbuf
                scatter_wait(b2)
                gather_start(b2, j + la)

        for j in range(la):
            gather_start(j % nbuf, j)
        for j in range(nbuf - la):
            b2 = (j + la) % nbuf
            gather_wait(j % nbuf)
            scatter_start(j % nbuf, j)
            gather_start(b2, j + la)
        start = nbuf - la
        ngroups = (nch - la - start) // nbuf

        def group(gi, carry):
            j0 = start + gi * nbuf
            for t in range(nbuf):
                do_chunk((start + t) % nbuf, j0 + t, True)
            return carry

        lax.fori_loop(0, ngroups, group, 0)
        for j in range(nch - la, nch):
            do_chunk(j % nbuf, j, False)
        for b in range(nbuf):
            scatter_wait(b)

        plsc.subcore_barrier()
        for t in range(4):
            pltpu.sync_copy(acc.at[pl.ds(s * n_per_s + t * q, q)], bounce)
            pltpu.sync_copy(
                bounce,
                out_hbm.at[pl.ds(c * n_pad + s * n_per_s + t * q, q)])

    return body(g, src2d, dst2d, zeros_nd)


def _dis_from_counts(c_blk):
    deg = c_blk[0] + c_blk[1] + 1.0
    return lax.rsqrt(deg)


def _tc_scale_matmul(x0, w1, counts2):
    n, d = x0.shape
    r = 1000
    grid = n // r

    def body(x_ref, w_ref, c_ref, g_ref):
        dis = _dis_from_counts(c_ref[...])
        h = lax.dot_general(x_ref[...], w_ref[...],
                            (((1,), (1,)), ((), ())),
                            preferred_element_type=jnp.float32)
        g_ref[...] = (h * dis).astype(jnp.bfloat16)

    return pl.pallas_call(
        body,
        out_shape=jax.ShapeDtypeStruct((n, d), jnp.bfloat16),
        grid=(grid,),
        in_specs=[
            pl.BlockSpec((r, d), lambda i: (i, 0)),
            pl.BlockSpec((d, d), lambda i: (0, 0)),
            pl.BlockSpec((2, r, 1), lambda i: (0, i, 0)),
        ],
        out_specs=pl.BlockSpec((r, d), lambda i: (i, 0)),
    )(x0, w1, counts2)


def _tc_finish_matmul(agg2, g1, counts2, b1, w2):
    n, d = g1.shape
    r = 1000
    grid = n // r

    def body(a_ref, g_ref, c_ref, b_ref, w_ref, h1_ref, g2_ref):
        dis = _dis_from_counts(c_ref[...])
        a = (a_ref[0].astype(jnp.float32) + a_ref[1].astype(jnp.float32)
             + g_ref[...].astype(jnp.float32))
        h1 = a * dis + b_ref[...]
        h1_ref[...] = h1
        h2l = lax.dot_general(h1, w_ref[...],
                              (((1,), (1,)), ((), ())),
                              preferred_element_type=jnp.float32)
        g2_ref[...] = (h2l * dis).astype(jnp.bfloat16)

    return pl.pallas_call(
        body,
        out_shape=(jax.ShapeDtypeStruct((n, d), jnp.float32),
                   jax.ShapeDtypeStruct((n, d), jnp.bfloat16)),
        grid=(grid,),
        in_specs=[
            pl.BlockSpec((2, r, d), lambda i: (0, i, 0)),
            pl.BlockSpec((r, d), lambda i: (i, 0)),
            pl.BlockSpec((2, r, 1), lambda i: (0, i, 0)),
            pl.BlockSpec((1, d), lambda i: (0, 0)),
            pl.BlockSpec((d, d), lambda i: (0, 0)),
        ],
        out_specs=(pl.BlockSpec((r, d), lambda i: (i, 0)),
                   pl.BlockSpec((r, d), lambda i: (i, 0))),
    )(agg2, g1, counts2, b1, w2)


def _tc_final(agg2, g2, counts2, b2, x0, h1):
    n, d = g2.shape
    r = 1000
    grid = n // r

    def body(a_ref, g_ref, c_ref, b_ref, x_ref, h1_ref, o_ref):
        dis = _dis_from_counts(c_ref[...])
        a = (a_ref[0].astype(jnp.float32) + a_ref[1].astype(jnp.float32)
             + g_ref[...].astype(jnp.float32))
        h2 = a * dis + b_ref[...]
        o_ref[...] = (x_ref[...] + h1_ref[...] + h2) * (1.0 / 3.0)

    return pl.pallas_call(
        body,
        out_shape=jax.ShapeDtypeStruct((n, d), jnp.float32),
        grid=(grid,),
        in_specs=[
            pl.BlockSpec((2, r, d), lambda i: (0, i, 0)),
            pl.BlockSpec((r, d), lambda i: (i, 0)),
            pl.BlockSpec((2, r, 1), lambda i: (0, i, 0)),
            pl.BlockSpec((1, d), lambda i: (0, 0)),
            pl.BlockSpec((r, d), lambda i: (i, 0)),
            pl.BlockSpec((r, d), lambda i: (i, 0)),
        ],
        out_specs=pl.BlockSpec((r, d), lambda i: (i, 0)),
    )(agg2, g2, counts2, b2, x0, h1)


def kernel(edge_index, u_emb, i_emb, W1, b1, W2, b2):
    n = u_emb.shape[0] + i_emb.shape[0]
    d = u_emb.shape[1]
    num_edges = edge_index.shape[1]
    ns = 16
    n_pad = ((n + 8 * ns - 1) // (8 * ns)) * (8 * ns)

    k = 80
    k2 = 125
    dst2d = edge_index[1].reshape(num_edges // k, k)
    src2d2 = edge_index[0].reshape(num_edges // k2, k2)
    dst2d2 = edge_index[1].reshape(num_edges // k2, k2)
    x0 = jnp.concatenate([u_emb, i_emb], axis=0)
    b1r = b1.reshape(1, d)
    b2r = b2.reshape(1, d)
    zeros_n = jnp.zeros((n_pad,), jnp.float32)
    zeros_nd = jnp.zeros((n_pad // 16, d), jnp.bfloat16)

    counts = _sc_degree_counts(dst2d, zeros_n, n_pad, num_edges)
    counts2 = counts.reshape(2, n_pad, 1)

    g1 = _tc_scale_matmul(x0, W1, counts2)
    agg1 = _sc_scatter_rows(g1, src2d2, dst2d2, zeros_nd, n_pad, num_edges)
    agg1 = agg1.reshape(2, n_pad, d)

    h1, g2 = _tc_finish_matmul(agg1, g1, counts2, b1r, W2)
    agg2 = _sc_scatter_rows(g2, src2d2, dst2d2, zeros_nd, n_pad, num_edges)
    agg2 = agg2.reshape(2, n_pad, d)

    return _tc_final(agg2, g2, counts2, b2r, x0, h1)
